# baseline (speedup 1.0000x reference)
import functools

import jax
import jax.numpy as jnp
from jax import lax
from jax.experimental import pallas as pl
from jax.experimental.pallas import tpu as pltpu
from jax.experimental.pallas import tpu_sc as plsc

B, K, D = 8, 2048, 256
RB = 256
R = B * K
NW = 32
RPW = R // NW
CH = 128
NCH = RPW // CH


def _deg_kernel(adj_ref, out_ref):
    x = adj_ref[0]
    acc = x[:, 0:128]
    for t in range(1, 16):
        acc = acc + x[:, t * 128:(t + 1) * 128]
    g = acc[:, 0:8]
    for u in range(1, 16):
        g = g + acc[:, u * 8:(u + 1) * 8]
    e04 = g[:, 0:1] + g[:, 4:5]
    e26 = g[:, 2:3] + g[:, 6:7]
    e15 = g[:, 1:2] + g[:, 5:6]
    e37 = g[:, 3:4] + g[:, 7:8]
    out_ref[0] = (e04 + e26) + (e15 + e37)


def _rank_kernel(deg_ref, rev_ref, sidx_ref):
    b = pl.program_id(0)
    r = pl.program_id(1)
    d = deg_ref[0, 0, :]
    di = deg_ref[0, 0, pl.ds(r * RB, RB)]
    dj2 = d[None, :]
    di2 = di[:, None]
    jj = lax.broadcasted_iota(jnp.int32, (RB, K), 1)
    ii = r * RB + lax.broadcasted_iota(jnp.int32, (RB, K), 0)
    m = (dj2 > di2) | ((dj2 == di2) & (jj < ii))
    rank = jnp.sum(m.astype(jnp.float32), axis=1).astype(jnp.int32)
    rev_ref[0, :, 0] = rank
    sidx_ref[0, :, 0] = rank + b * K


@functools.lru_cache(maxsize=1)
def _make_sc_scatter():
    @functools.partial(
        pl.kernel,
        mesh=plsc.VectorSubcoreMesh(core_axis_name="c", subcore_axis_name="s",
                                    num_cores=2, num_subcores=16),
        out_type=[
            jax.ShapeDtypeStruct((R,), jnp.int32),
            jax.ShapeDtypeStruct((R, D), jnp.float32),
        ],
        scratch_types=[
            pltpu.VMEM((NCH, CH), jnp.int32),
            pltpu.VMEM((NCH, CH), jnp.int32),
            pltpu.VMEM((CH, D), jnp.float32),
            pltpu.VMEM((CH, D), jnp.float32),
            pltpu.SemaphoreType.DMA,
            pltpu.SemaphoreType.DMA,
        ],
    )
    def _sc_scatter(slots_hbm, sidx_hbm, rowid_hbm, order_hbm, ordered_hbm,
                    idx_v, val_v, rows0, rows1, sem0, sem1):
        wid = lax.axis_index("s") * 2 + lax.axis_index("c")
        base = wid * RPW
        for j in range(NCH):
            pltpu.sync_copy(sidx_hbm.at[pl.ds(base + j * CH, CH)], idx_v.at[j])
            pltpu.sync_copy(rowid_hbm.at[pl.ds(base + j * CH, CH)], val_v.at[j])
        for j in range(NCH):
            pltpu.async_copy(val_v.at[j], order_hbm.at[idx_v.at[j]], sem0).wait()
        bufs = (rows0, rows1)
        for j in range(NCH):
            buf = bufs[j % 2]
            pltpu.sync_copy(slots_hbm.at[pl.ds(base + j * CH, CH)], buf)
            pltpu.async_copy(buf, ordered_hbm.at[idx_v.at[j]], sem1).wait()

    return _sc_scatter


def kernel(slots, adj):
    deg = pl.pallas_call(
        _deg_kernel,
        grid=(B, K // RB),
        in_specs=[pl.BlockSpec((1, RB, K), lambda b, r: (b, r, 0))],
        out_specs=pl.BlockSpec((1, RB, 1), lambda b, r: (b * (K // RB) + r, 0, 0)),
        out_shape=jax.ShapeDtypeStruct((B * K // RB, RB, 1), jnp.float32),
    )(adj)

    rev, sidx = pl.pallas_call(
        _rank_kernel,
        grid=(B, K // RB),
        in_specs=[pl.BlockSpec((1, 1, K), lambda b, r: (b, 0, 0))],
        out_specs=[
            pl.BlockSpec((1, RB, 1), lambda b, r: (b * (K // RB) + r, 0, 0)),
            pl.BlockSpec((1, RB, 1), lambda b, r: (b * (K // RB) + r, 0, 0)),
        ],
        out_shape=[
            jax.ShapeDtypeStruct((B * K // RB, RB, 1), jnp.int32),
            jax.ShapeDtypeStruct((B * K // RB, RB, 1), jnp.int32),
        ],
    )(deg.reshape(B, 1, K))

    slots_flat = slots.reshape(R, D)
    sidx_flat = sidx.reshape(R)
    rowid = jnp.tile(jnp.arange(K, dtype=jnp.int32), B)

    order_flat, ordered_flat = _make_sc_scatter()(slots_flat, sidx_flat, rowid)

    return (ordered_flat.reshape(B, K, D),
            order_flat.reshape(B, K),
            rev.reshape(B, K))


# speedup vs baseline: 1.3297x; 1.3297x over previous
import functools

import jax
import jax.numpy as jnp
from jax import lax
from jax.experimental import pallas as pl
from jax.experimental.pallas import tpu as pltpu
from jax.experimental.pallas import tpu_sc as plsc

B, K, D = 8, 2048, 256
RB = 256
NBLK = K // RB
R = B * K
NW = 32
RPW = R // NW
CH = 128
NCH = RPW // CH
NBUF = 3


def _fused_kernel(adj_ref, rev_ref, sidx_ref, deg_scr):
    s = pl.program_id(0)
    t = pl.program_id(1)

    @pl.when(s < B)
    def _deg():
        x = adj_ref[0]
        acc = x[:, 0:128]
        for c in range(1, 16):
            acc = acc + x[:, c * 128:(c + 1) * 128]
        tr = acc.T
        g = tr[0:8, :]
        for u in range(1, 16):
            g = g + tr[u * 8:(u + 1) * 8, :]
        e04 = g[0:1, :] + g[4:5, :]
        e26 = g[2:3, :] + g[6:7, :]
        e15 = g[1:2, :] + g[5:6, :]
        e37 = g[3:4, :] + g[7:8, :]
        deg_scr[s % 2, pl.ds(t * RB, RB)] = ((e04 + e26) + (e15 + e37))[0]

    @pl.when(s > 0)
    def _rank():
        row = (s - 1) % 2
        d = deg_scr[row, :][None, :]
        di = deg_scr[row, pl.ds(t * RB, RB)][:, None]
        jj = lax.broadcasted_iota(jnp.int32, (RB, K), 1)
        ii = t * RB + lax.broadcasted_iota(jnp.int32, (RB, K), 0)
        lt = (jj < ii).astype(jnp.float32)
        gt = (d > di).astype(jnp.float32)
        m = jnp.where(d == di, lt, gt)
        rank = jnp.sum(m, axis=1).astype(jnp.int32)
        rev_ref[0, :, 0] = rank
        sidx_ref[0, :, 0] = rank + (s - 1) * K


@functools.lru_cache(maxsize=1)
def _make_sc_scatter():
    @functools.partial(
        pl.kernel,
        mesh=plsc.VectorSubcoreMesh(core_axis_name="c", subcore_axis_name="s",
                                    num_cores=2, num_subcores=16),
        out_type=[
            jax.ShapeDtypeStruct((R,), jnp.int32),
            jax.ShapeDtypeStruct((R, D), jnp.float32),
        ],
        scratch_types=[
            pltpu.VMEM((NCH, CH), jnp.int32),
            pltpu.VMEM((NCH, CH), jnp.int32),
            pltpu.VMEM((CH, D), jnp.float32),
            pltpu.VMEM((CH, D), jnp.float32),
            pltpu.VMEM((CH, D), jnp.float32),
            pltpu.SemaphoreType.DMA,
            pltpu.SemaphoreType.DMA,
            pltpu.SemaphoreType.DMA,
            pltpu.SemaphoreType.DMA,
        ],
    )
    def _sc_scatter(slots_hbm, sidx_hbm, rowid_hbm, order_hbm, ordered_hbm,
                    idx_v, val_v, b0, b1, b2, sem_i, sem_o, sem_l, sem_s):
        wid = lax.axis_index("s") * 2 + lax.axis_index("c")
        base = wid * RPW
        ci = pltpu.async_copy(sidx_hbm.at[wid], idx_v, sem_i)
        cv = pltpu.async_copy(rowid_hbm.at[wid], val_v, sem_i)
        ci.wait()
        cv.wait()
        ocs = [pltpu.async_copy(val_v.at[j], order_hbm.at[idx_v.at[j]], sem_o)
               for j in range(NCH)]
        bufs = (b0, b1, b2)
        loads = [None] * NCH
        scats = [None] * NCH
        for j in range(min(NBUF, NCH)):
            loads[j] = pltpu.async_copy(
                slots_hbm.at[pl.ds(base + j * CH, CH)], bufs[j % NBUF], sem_l)
        for j in range(NCH):
            loads[j].wait()
            scats[j] = pltpu.async_copy(
                bufs[j % NBUF], ordered_hbm.at[idx_v.at[j]], sem_s)
            nj = j + NBUF
            if nj < NCH:
                scats[nj - NBUF].wait()
                loads[nj] = pltpu.async_copy(
                    slots_hbm.at[pl.ds(base + nj * CH, CH)], bufs[nj % NBUF],
                    sem_l)
        for j in range(NCH):
            if scats[j] is not None and (j + NBUF >= NCH):
                scats[j].wait()
        for oc in ocs:
            oc.wait()

    return _sc_scatter


def kernel(slots, adj):
    rev, sidx = pl.pallas_call(
        _fused_kernel,
        grid=(B + 1, NBLK),
        in_specs=[pl.BlockSpec(
            (1, RB, K), lambda s, t: (jnp.minimum(s, B - 1), t, 0))],
        out_specs=[
            pl.BlockSpec((1, RB, 1),
                         lambda s, t: (jnp.maximum(s - 1, 0) * NBLK + t, 0, 0)),
            pl.BlockSpec((1, RB, 1),
                         lambda s, t: (jnp.maximum(s - 1, 0) * NBLK + t, 0, 0)),
        ],
        out_shape=[
            jax.ShapeDtypeStruct((B * NBLK, RB, 1), jnp.int32),
            jax.ShapeDtypeStruct((B * NBLK, RB, 1), jnp.int32),
        ],
        scratch_shapes=[pltpu.VMEM((2, K), jnp.float32)],
    )(adj)

    slots_flat = slots.reshape(R, D)
    sidx3 = sidx.reshape(NW, NCH, CH)
    rowid = jnp.tile(jnp.arange(K, dtype=jnp.int32), B).reshape(NW, NCH, CH)

    order_flat, ordered_flat = _make_sc_scatter()(slots_flat, sidx3, rowid)

    return (ordered_flat.reshape(B, K, D),
            order_flat.reshape(B, K),
            rev.reshape(B, K))


# speedup vs baseline: 1.5839x; 1.1912x over previous
import functools

import jax
import jax.numpy as jnp
from jax import lax
from jax.experimental import pallas as pl
from jax.experimental.pallas import tpu as pltpu
from jax.experimental.pallas import tpu_sc as plsc

B, K, D = 8, 2048, 256
RB = 256
NBLK = K // RB
R = B * K
NW = 32
RPW = R // NW
HALF = R // 2
CH = 64
NCH = RPW // CH
NBUF = 6


def _fused_kernel(adj_ref, rev_ref, sidx_ref, lidx_ref, deg_scr):
    s = pl.program_id(0)
    t = pl.program_id(1)

    @pl.when(s < B)
    def _deg():
        x = adj_ref[0]
        acc = x[:, 0:128]
        for c in range(1, 16):
            acc = acc + x[:, c * 128:(c + 1) * 128]
        tr = acc.T
        g = tr[0:8, :]
        for u in range(1, 16):
            g = g + tr[u * 8:(u + 1) * 8, :]
        e04 = g[0:1, :] + g[4:5, :]
        e26 = g[2:3, :] + g[6:7, :]
        e15 = g[1:2, :] + g[5:6, :]
        e37 = g[3:4, :] + g[7:8, :]
        deg_scr[s % 2, pl.ds(t * RB, RB)] = ((e04 + e26) + (e15 + e37))[0]

    @pl.when(s > 0)
    def _rank():
        row = (s - 1) % 2
        d = deg_scr[row, :][None, :]
        di = deg_scr[row, pl.ds(t * RB, RB)][:, None]
        jj = lax.broadcasted_iota(jnp.int32, (RB, K), 1)
        ii = t * RB + lax.broadcasted_iota(jnp.int32, (RB, K), 0)
        lt = (jj < ii).astype(jnp.float32)
        gt = (d > di).astype(jnp.float32)
        m = jnp.where(d == di, lt, gt)
        rank = jnp.sum(m, axis=1).astype(jnp.int32)
        rev_ref[0, :, 0] = rank
        sidx_ref[0, :, 0] = rank + (s - 1) * K
        lidx_ref[0, :, 0] = rank + lax.rem(s - 1, 4) * K


@functools.lru_cache(maxsize=1)
def _make_sc_scatter():
    @functools.partial(
        pl.kernel,
        mesh=plsc.VectorSubcoreMesh(core_axis_name="c", subcore_axis_name="s",
                                    num_cores=2, num_subcores=16),
        out_type=[
            jax.ShapeDtypeStruct((R,), jnp.int32),
            jax.ShapeDtypeStruct((R, D), jnp.float32),
        ],
        scratch_types=[
            pltpu.VMEM((NCH, CH), jnp.int32),
            pltpu.VMEM((NCH, CH), jnp.int32),
            pltpu.VMEM((NCH, CH), jnp.int32),
            pltpu.VMEM((RPW,), jnp.int32),
            pltpu.VMEM_SHARED((HALF,), jnp.int32),
            pltpu.VMEM((CH, D), jnp.float32),
            pltpu.VMEM((CH, D), jnp.float32),
            pltpu.VMEM((CH, D), jnp.float32),
            pltpu.VMEM((CH, D), jnp.float32),
            pltpu.VMEM((CH, D), jnp.float32),
            pltpu.VMEM((CH, D), jnp.float32),
            pltpu.SemaphoreType.DMA,
            pltpu.SemaphoreType.DMA,
            pltpu.SemaphoreType.DMA,
        ],
    )
    def _sc_scatter(slots_hbm, sidx_hbm, lidx_hbm, rowid_hbm,
                    order_hbm, ordered_hbm,
                    idx_v, lidx_v, val_v, oslice_v, order_sh,
                    b0, b1, b2, b3, b4, b5, sem_i, sem_l, sem_s):
        cid = lax.axis_index("c")
        sid = lax.axis_index("s")
        wid = cid * 16 + sid
        base = wid * RPW
        ci = pltpu.async_copy(sidx_hbm.at[wid], idx_v, sem_i)
        cl = pltpu.async_copy(lidx_hbm.at[wid], lidx_v, sem_i)
        cv = pltpu.async_copy(rowid_hbm.at[wid], val_v, sem_i)
        ci.wait()
        cl.wait()
        cv.wait()
        for j in range(NCH):
            pltpu.sync_copy(val_v.at[j], order_sh.at[lidx_v.at[j]])
        bufs = (b0, b1, b2, b3, b4, b5)
        loads = [None] * NCH
        scats = [None] * NCH
        for j in range(min(NBUF, NCH)):
            loads[j] = pltpu.async_copy(
                slots_hbm.at[pl.ds(base + j * CH, CH)], bufs[j % NBUF], sem_l)
        for j in range(NCH):
            loads[j].wait()
            scats[j] = pltpu.async_copy(
                bufs[j % NBUF], ordered_hbm.at[idx_v.at[j]], sem_s)
            nj = j + NBUF
            if nj < NCH:
                scats[nj - NBUF].wait()
                loads[nj] = pltpu.async_copy(
                    slots_hbm.at[pl.ds(base + nj * CH, CH)], bufs[nj % NBUF],
                    sem_l)
        plsc.subcore_barrier()
        pltpu.sync_copy(order_sh.at[pl.ds(sid * RPW, RPW)], oslice_v)
        pltpu.sync_copy(oslice_v, order_hbm.at[pl.ds(base, RPW)])
        for j in range(NCH):
            if scats[j] is not None and (j + NBUF >= NCH):
                scats[j].wait()

    return _sc_scatter


def kernel(slots, adj):
    rev, sidx, lidx = pl.pallas_call(
        _fused_kernel,
        grid=(B + 1, NBLK),
        in_specs=[pl.BlockSpec(
            (1, RB, K),
            lambda s, t: (jnp.where(s < B, s, B - 1),
                          jnp.where(s < B, t, NBLK - 1), 0))],
        out_specs=[
            pl.BlockSpec((1, RB, 1),
                         lambda s, t: (jnp.maximum(s - 1, 0) * NBLK + t, 0, 0)),
            pl.BlockSpec((1, RB, 1),
                         lambda s, t: (jnp.maximum(s - 1, 0) * NBLK + t, 0, 0)),
            pl.BlockSpec((1, RB, 1),
                         lambda s, t: (jnp.maximum(s - 1, 0) * NBLK + t, 0, 0)),
        ],
        out_shape=[
            jax.ShapeDtypeStruct((B * NBLK, RB, 1), jnp.int32),
            jax.ShapeDtypeStruct((B * NBLK, RB, 1), jnp.int32),
            jax.ShapeDtypeStruct((B * NBLK, RB, 1), jnp.int32),
        ],
        scratch_shapes=[pltpu.VMEM((2, K), jnp.float32)],
    )(adj)

    slots_flat = slots.reshape(R, D)
    sidx3 = sidx.reshape(NW, NCH, CH)
    lidx3 = lidx.reshape(NW, NCH, CH)
    rowid = jnp.tile(jnp.arange(K, dtype=jnp.int32), B).reshape(NW, NCH, CH)

    order_flat, ordered_flat = _make_sc_scatter()(
        slots_flat, sidx3, lidx3, rowid)

    return (ordered_flat.reshape(B, K, D),
            order_flat.reshape(B, K),
            rev.reshape(B, K))


# speedup vs baseline: 2.0248x; 1.2784x over previous
import functools

import jax
import jax.numpy as jnp
from jax import lax
from jax.experimental import pallas as pl
from jax.experimental.pallas import tpu as pltpu
from jax.experimental.pallas import tpu_sc as plsc

B, K, D = 8, 2048, 256
RB = 1024
NBLK = K // RB
R = B * K
NW = 32
RPW = R // NW
HALF = R // 2
CH = 64
NCH = RPW // CH
NBUF = 6


def _fused_kernel(adj_ref, rev_ref, sidx_ref, lidx_ref, deg_scr):
    s = pl.program_id(0)
    t = pl.program_id(1)

    @pl.when(s < B)
    def _deg():
        x = adj_ref[0]
        acc = x[:, 0:128]
        for c in range(1, 16):
            acc = acc + x[:, c * 128:(c + 1) * 128]
        tr = acc.T
        g = tr[0:8, :]
        for u in range(1, 16):
            g = g + tr[u * 8:(u + 1) * 8, :]
        e04 = g[0:1, :] + g[4:5, :]
        e26 = g[2:3, :] + g[6:7, :]
        e15 = g[1:2, :] + g[5:6, :]
        e37 = g[3:4, :] + g[7:8, :]
        deg_scr[s % 2, pl.ds(t * RB, RB)] = ((e04 + e26) + (e15 + e37))[0]

    @pl.when(s > 0)
    def _rank():
        row = (s - 1) % 2
        d = deg_scr[row, :][None, :]
        di = deg_scr[row, pl.ds(t * RB, RB)][:, None]
        jj = lax.broadcasted_iota(jnp.int32, (RB, K), 1)
        ii = t * RB + lax.broadcasted_iota(jnp.int32, (RB, K), 0)
        lt = (jj < ii).astype(jnp.float32)
        gt = (d > di).astype(jnp.float32)
        m = jnp.where(d == di, lt, gt)
        rank = jnp.sum(m, axis=1).astype(jnp.int32)
        rev_ref[0, :, 0] = rank
        sidx_ref[0, :, 0] = rank + (s - 1) * K
        lidx_ref[0, :, 0] = rank + lax.rem(s - 1, 4) * K


@functools.lru_cache(maxsize=1)
def _make_sc_scatter():
    @functools.partial(
        pl.kernel,
        mesh=plsc.VectorSubcoreMesh(core_axis_name="c", subcore_axis_name="s",
                                    num_cores=2, num_subcores=16),
        out_type=[
            jax.ShapeDtypeStruct((R,), jnp.int32),
            jax.ShapeDtypeStruct((R, D), jnp.float32),
        ],
        scratch_types=[
            pltpu.VMEM((NCH, CH), jnp.int32),
            pltpu.VMEM((NCH, CH), jnp.int32),
            pltpu.VMEM((NCH, CH), jnp.int32),
            pltpu.VMEM((RPW,), jnp.int32),
            pltpu.VMEM_SHARED((HALF,), jnp.int32),
            pltpu.VMEM((CH, D), jnp.float32),
            pltpu.VMEM((CH, D), jnp.float32),
            pltpu.VMEM((CH, D), jnp.float32),
            pltpu.VMEM((CH, D), jnp.float32),
            pltpu.VMEM((CH, D), jnp.float32),
            pltpu.VMEM((CH, D), jnp.float32),
            pltpu.SemaphoreType.DMA,
            pltpu.SemaphoreType.DMA,
            pltpu.SemaphoreType.DMA,
        ],
    )
    def _sc_scatter(slots_hbm, sidx_hbm, lidx_hbm, rowid_hbm,
                    order_hbm, ordered_hbm,
                    idx_v, lidx_v, val_v, oslice_v, order_sh,
                    b0, b1, b2, b3, b4, b5, sem_i, sem_l, sem_s):
        cid = lax.axis_index("c")
        sid = lax.axis_index("s")
        wid = cid * 16 + sid
        base = wid * RPW
        ci = pltpu.async_copy(sidx_hbm.at[wid], idx_v, sem_i)
        cl = pltpu.async_copy(lidx_hbm.at[wid], lidx_v, sem_i)
        cv = pltpu.async_copy(rowid_hbm.at[wid], val_v, sem_i)
        ci.wait()
        cl.wait()
        cv.wait()
        for j in range(NCH):
            pltpu.sync_copy(val_v.at[j], order_sh.at[lidx_v.at[j]])
        bufs = (b0, b1, b2, b3, b4, b5)
        loads = [None] * NCH
        scats = [None] * NCH
        for j in range(min(NBUF, NCH)):
            loads[j] = pltpu.async_copy(
                slots_hbm.at[pl.ds(base + j * CH, CH)], bufs[j % NBUF], sem_l)
        for j in range(NCH):
            loads[j].wait()
            scats[j] = pltpu.async_copy(
                bufs[j % NBUF], ordered_hbm.at[idx_v.at[j]], sem_s)
            nj = j + NBUF
            if nj < NCH:
                scats[nj - NBUF].wait()
                loads[nj] = pltpu.async_copy(
                    slots_hbm.at[pl.ds(base + nj * CH, CH)], bufs[nj % NBUF],
                    sem_l)
        plsc.subcore_barrier()
        pltpu.sync_copy(order_sh.at[pl.ds(sid * RPW, RPW)], oslice_v)
        pltpu.sync_copy(oslice_v, order_hbm.at[pl.ds(base, RPW)])
        for j in range(NCH):
            if scats[j] is not None and (j + NBUF >= NCH):
                scats[j].wait()

    return _sc_scatter


def kernel(slots, adj):
    rev, sidx, lidx = pl.pallas_call(
        _fused_kernel,
        grid=(B + 1, NBLK),
        in_specs=[pl.BlockSpec(
            (1, RB, K),
            lambda s, t: (jnp.where(s < B, s, B - 1),
                          jnp.where(s < B, t, NBLK - 1), 0))],
        out_specs=[
            pl.BlockSpec((1, RB, 1),
                         lambda s, t: (jnp.maximum(s - 1, 0) * NBLK + t, 0, 0)),
            pl.BlockSpec((1, RB, 1),
                         lambda s, t: (jnp.maximum(s - 1, 0) * NBLK + t, 0, 0)),
            pl.BlockSpec((1, RB, 1),
                         lambda s, t: (jnp.maximum(s - 1, 0) * NBLK + t, 0, 0)),
        ],
        out_shape=[
            jax.ShapeDtypeStruct((B * NBLK, RB, 1), jnp.int32),
            jax.ShapeDtypeStruct((B * NBLK, RB, 1), jnp.int32),
            jax.ShapeDtypeStruct((B * NBLK, RB, 1), jnp.int32),
        ],
        scratch_shapes=[pltpu.VMEM((2, K), jnp.float32)],
    )(adj)

    slots_flat = slots.reshape(R, D)
    sidx3 = sidx.reshape(NW, NCH, CH)
    lidx3 = lidx.reshape(NW, NCH, CH)
    rowid = jnp.tile(jnp.arange(K, dtype=jnp.int32), B).reshape(NW, NCH, CH)

    order_flat, ordered_flat = _make_sc_scatter()(
        slots_flat, sidx3, lidx3, rowid)

    return (ordered_flat.reshape(B, K, D),
            order_flat.reshape(B, K),
            rev.reshape(B, K))


# speedup vs baseline: 2.0717x; 1.0232x over previous
import functools

import jax
import jax.numpy as jnp
from jax import lax
from jax.experimental import pallas as pl
from jax.experimental.pallas import tpu as pltpu
from jax.experimental.pallas import tpu_sc as plsc

B, K, D = 8, 2048, 256
RB = 1024
NBLK = K // RB
R = B * K
NW = 32
RPW = R // NW
HALF = R // 2
CH = 64
NCH = RPW // CH
NBUF = 6


def _fused_kernel(adj_ref, tri_ref, rev_ref, sidx_ref, lidx_ref, deg_scr,
                  deg_scr_t):
    s = pl.program_id(0)
    t = pl.program_id(1)

    @pl.when(s < B)
    def _deg():
        x = adj_ref[0]
        acc = x[:, 0:128]
        for c in range(1, 16):
            acc = acc + x[:, c * 128:(c + 1) * 128]
        tr = acc.T
        g = tr[0:8, :]
        for u in range(1, 16):
            g = g + tr[u * 8:(u + 1) * 8, :]
        e04 = g[0:1, :] + g[4:5, :]
        e26 = g[2:3, :] + g[6:7, :]
        e15 = g[1:2, :] + g[5:6, :]
        e37 = g[3:4, :] + g[7:8, :]
        tot = (e04 + e26) + (e15 + e37)
        deg_scr[s % 2, pl.ds(t * RB, RB)] = tot[0]
        deg_scr_t[s % 2, pl.ds(t * RB, RB), 0] = tot.T[:, 0]

    @pl.when(s > 0)
    def _rank():
        row = (s - 1) % 2
        dA = deg_scr[row, 0:RB][None, :]
        dB = deg_scr[row, RB:K][None, :]
        di = deg_scr_t[row, pl.ds(t * RB, RB), :]
        tri = tri_ref[0]

        @pl.when(t == 0)
        def _():
            mA = jnp.where(dA == di, tri, (dA > di).astype(jnp.float32))
            mB = (dB > di).astype(jnp.float32)
            rank = (jnp.sum(mA, axis=1) + jnp.sum(mB, axis=1)).astype(jnp.int32)
            rev_ref[0, :, 0] = rank
            sidx_ref[0, :, 0] = rank + (s - 1) * K
            lidx_ref[0, :, 0] = rank + lax.rem(s - 1, 4) * K

        @pl.when(t == 1)
        def _():
            mA = (dA >= di).astype(jnp.float32)
            mB = jnp.where(dB == di, tri, (dB > di).astype(jnp.float32))
            rank = (jnp.sum(mA, axis=1) + jnp.sum(mB, axis=1)).astype(jnp.int32)
            rev_ref[0, :, 0] = rank
            sidx_ref[0, :, 0] = rank + (s - 1) * K
            lidx_ref[0, :, 0] = rank + lax.rem(s - 1, 4) * K


@functools.lru_cache(maxsize=1)
def _make_sc_scatter():
    @functools.partial(
        pl.kernel,
        mesh=plsc.VectorSubcoreMesh(core_axis_name="c", subcore_axis_name="s",
                                    num_cores=2, num_subcores=16),
        out_type=[
            jax.ShapeDtypeStruct((R,), jnp.int32),
            jax.ShapeDtypeStruct((R, D), jnp.float32),
        ],
        scratch_types=[
            pltpu.VMEM((NCH, CH), jnp.int32),
            pltpu.VMEM((NCH, CH), jnp.int32),
            pltpu.VMEM((NCH, CH), jnp.int32),
            pltpu.VMEM((RPW,), jnp.int32),
            pltpu.VMEM_SHARED((HALF,), jnp.int32),
            pltpu.VMEM((CH, D), jnp.float32),
            pltpu.VMEM((CH, D), jnp.float32),
            pltpu.VMEM((CH, D), jnp.float32),
            pltpu.VMEM((CH, D), jnp.float32),
            pltpu.VMEM((CH, D), jnp.float32),
            pltpu.VMEM((CH, D), jnp.float32),
            pltpu.SemaphoreType.DMA,
            pltpu.SemaphoreType.DMA,
            pltpu.SemaphoreType.DMA,
        ],
    )
    def _sc_scatter(slots_hbm, sidx_hbm, lidx_hbm, rowid_hbm,
                    order_hbm, ordered_hbm,
                    idx_v, lidx_v, val_v, oslice_v, order_sh,
                    b0, b1, b2, b3, b4, b5, sem_i, sem_l, sem_s):
        cid = lax.axis_index("c")
        sid = lax.axis_index("s")
        wid = cid * 16 + sid
        base = wid * RPW
        ci = pltpu.async_copy(sidx_hbm.at[wid], idx_v, sem_i)
        cl = pltpu.async_copy(lidx_hbm.at[wid], lidx_v, sem_i)
        cv = pltpu.async_copy(rowid_hbm.at[wid], val_v, sem_i)
        ci.wait()
        cl.wait()
        cv.wait()
        for j in range(NCH):
            pltpu.sync_copy(val_v.at[j], order_sh.at[lidx_v.at[j]])
        bufs = (b0, b1, b2, b3, b4, b5)
        loads = [None] * NCH
        scats = [None] * NCH
        for j in range(min(NBUF, NCH)):
            loads[j] = pltpu.async_copy(
                slots_hbm.at[pl.ds(base + j * CH, CH)], bufs[j % NBUF], sem_l)
        for j in range(NCH):
            loads[j].wait()
            scats[j] = pltpu.async_copy(
                bufs[j % NBUF], ordered_hbm.at[idx_v.at[j]], sem_s)
            nj = j + NBUF
            if nj < NCH:
                scats[nj - NBUF].wait()
                loads[nj] = pltpu.async_copy(
                    slots_hbm.at[pl.ds(base + nj * CH, CH)], bufs[nj % NBUF],
                    sem_l)
        plsc.subcore_barrier()
        pltpu.sync_copy(order_sh.at[pl.ds(sid * RPW, RPW)], oslice_v)
        pltpu.sync_copy(oslice_v, order_hbm.at[pl.ds(base, RPW)])
        for j in range(NCH):
            if scats[j] is not None and (j + NBUF >= NCH):
                scats[j].wait()

    return _sc_scatter


@functools.lru_cache(maxsize=1)
def _tri():
    return jax.device_put(jnp.tril(jnp.ones((RB, RB), jnp.float32), -1)
                          .reshape(1, RB, RB))


def kernel(slots, adj):
    rev, sidx, lidx = pl.pallas_call(
        _fused_kernel,
        grid=(B + 1, NBLK),
        in_specs=[
            pl.BlockSpec(
                (1, RB, K),
                lambda s, t: (jnp.where(s < B, s, B - 1),
                              jnp.where(s < B, t, NBLK - 1), 0)),
            pl.BlockSpec((1, RB, RB), lambda s, t: (0, 0, 0)),
        ],
        out_specs=[
            pl.BlockSpec((1, RB, 1), lambda s, t: (s * NBLK + t, 0, 0)),
            pl.BlockSpec((1, RB, 1), lambda s, t: (s * NBLK + t, 0, 0)),
            pl.BlockSpec((1, RB, 1), lambda s, t: (s * NBLK + t, 0, 0)),
        ],
        out_shape=[
            jax.ShapeDtypeStruct(((B + 1) * NBLK, RB, 1), jnp.int32),
            jax.ShapeDtypeStruct(((B + 1) * NBLK, RB, 1), jnp.int32),
            jax.ShapeDtypeStruct(((B + 1) * NBLK, RB, 1), jnp.int32),
        ],
        scratch_shapes=[pltpu.VMEM((2, K), jnp.float32),
                        pltpu.VMEM((2, K, 1), jnp.float32)],
    )(adj, _tri())
    rev = rev[NBLK:]
    sidx = sidx[NBLK:]
    lidx = lidx[NBLK:]

    slots_flat = slots.reshape(R, D)
    sidx3 = sidx.reshape(NW, NCH, CH)
    lidx3 = lidx.reshape(NW, NCH, CH)
    rowid = jnp.tile(jnp.arange(K, dtype=jnp.int32), B).reshape(NW, NCH, CH)

    order_flat, ordered_flat = _make_sc_scatter()(
        slots_flat, sidx3, lidx3, rowid)

    return (ordered_flat.reshape(B, K, D),
            order_flat.reshape(B, K),
            rev.reshape(B, K))
